# baseline (speedup 1.0000x reference)
import jax
import jax.numpy as jnp
from jax import lax
from jax.experimental import pallas as pl
from jax.experimental.pallas import tpu as pltpu
from jax.experimental.pallas import tpu_sc as plsc

TOPK = 256
NIN = 1_000_000
WORKERS = 32
PER_W = 31_264
NPAD = WORKERS * PER_W
VECS = PER_W // 16
K1_WINDOWS = 2
WIN = PER_W // K1_WINDOWS
WVECS = WIN // 16
BINS = 4096
SHIFT = 20
CAP = 64
ROW = 128
SLOP = ROW + 16
C = WORKERS * CAP


def _ordered_key(v):
    b = plsc.bitcast(v, jnp.uint32)
    neg = b >= jnp.uint32(0x80000000)
    return jnp.where(neg, ~b, b | jnp.uint32(0x80000000))


def _wid():
    return lax.axis_index("c") * 16 + lax.axis_index("s")


def _k1_body(scores_hbm, maskf_hbm, ghist_out,
             hist, merged, colsum, win_s, win_m, sp_hist):
    c = lax.axis_index("c")
    s = lax.axis_index("s")
    base = _wid() * PER_W
    lane = lax.iota(jnp.int32, 16)
    ones = jnp.ones((16,), jnp.int32)
    zeros = jnp.zeros((16,), jnp.int32)

    def zero_j(j, _):
        for k in range(16):
            hist[pl.ds(j * 256 + k * 16, 16)] = zeros
        return 0
    lax.fori_loop(0, BINS // 16, zero_j, 0)

    def win_pass(w, _):
        off = base + w * WIN
        pltpu.sync_copy(scores_hbm.at[pl.ds(off, WIN)], win_s)
        pltpu.sync_copy(maskf_hbm.at[pl.ds(off, WIN)], win_m)

        def vec1(v, _):
            sv = win_s[pl.ds(v * 16, 16)]
            mv = win_m[pl.ds(v * 16, 16)]
            key = _ordered_key(sv * mv)
            flat = lax.convert_element_type(key >> jnp.uint32(SHIFT - 4),
                                            jnp.int32)
            flat = (flat & -16) | lane
            plsc.addupdate_scatter(hist, [flat], ones)
            return 0
        lax.fori_loop(0, WVECS, vec1, 0)
        return 0
    lax.fori_loop(0, K1_WINDOWS, win_pass, 0)

    idxbase = [lane * 16 + l for l in range(16)]

    def merge_j(j, _):
        acc = plsc.load_gather(hist, [idxbase[0] + j * 256])
        for l in range(1, 16):
            acc = acc + plsc.load_gather(hist, [idxbase[l] + j * 256])
        merged[pl.ds(j * 16, 16)] = acc
        return 0
    lax.fori_loop(0, BINS // 16, merge_j, 0)
    pltpu.sync_copy(merged, sp_hist.at[s])
    plsc.subcore_barrier()

    pltpu.sync_copy(sp_hist.at[:, pl.ds(s * 256, 256)], colsum)

    def col_j(j, _):
        acc = colsum[0, pl.ds(j * 16, 16)]
        for l in range(1, 16):
            acc = acc + colsum[l, pl.ds(j * 16, 16)]
        merged[pl.ds(j * 16, 16)] = acc
        return 0
    lax.fori_loop(0, 16, col_j, 0)
    pltpu.sync_copy(merged.at[pl.ds(0, 256)],
                    ghist_out.at[pl.ds(c * BINS + s * 256, 256)])


_k1_call = pl.kernel(
    _k1_body,
    out_type=jax.ShapeDtypeStruct((2 * BINS,), jnp.int32),
    mesh=plsc.VectorSubcoreMesh(
        core_axis_name="c", subcore_axis_name="s", num_cores=2),
    compiler_params=pltpu.CompilerParams(needs_layout_passes=False),
    scratch_types=[
        pltpu.VMEM((BINS * 16,), jnp.int32),
        pltpu.VMEM((BINS,), jnp.int32),
        pltpu.VMEM((16, 256), jnp.int32),
        pltpu.VMEM((WIN,), jnp.float32),
        pltpu.VMEM((WIN,), jnp.float32),
        pltpu.VMEM_SHARED((16, BINS), jnp.int32),
    ],
)


def _k2_body(scores_hbm, maskf_hbm, ghist_hbm, vals_out, idx_out, cnt_out,
             ghist, chunk_s, chunk_m, cand_v, cand_i, cnt_vec):
    base = _wid() * PER_W
    lane = lax.iota(jnp.int32, 16)
    zeros = jnp.zeros((16,), jnp.int32)

    pltpu.sync_copy(ghist_hbm, ghist)
    pltpu.sync_copy(scores_hbm.at[pl.ds(base, PER_W)], chunk_s)
    pltpu.sync_copy(maskf_hbm.at[pl.ds(base, PER_W)], chunk_m)

    def scan_i(i, carry):
        acc, done, tkey = carry
        j = (BINS // 16 - 1) - i
        g = (ghist[pl.ds(j * 16, 16)]
             + ghist[pl.ds(BINS + j * 16, 16)])
        rev = lax.rev(g, (0,))
        cs = lax.cumsum(rev)
        total = jnp.max(cs)
        ge = (acc + cs) >= TOPK
        f = jnp.max(plsc.all_reduce_ffs(ge))
        tj = lax.convert_element_type(j * 16 + 15 - f, jnp.uint32)
        tj = tj << jnp.uint32(SHIFT)
        crossed = (acc + total) >= TOPK
        first = jnp.logical_and(jnp.logical_not(done), crossed)
        tkey = jnp.where(first, tj, tkey)
        done = jnp.logical_or(done, crossed)
        return acc + total, done, tkey
    _, _, tkey = lax.fori_loop(
        0, BINS // 16, scan_i,
        (jnp.int32(0), jnp.bool_(False), jnp.uint32(0)))

    neg_inf = jnp.full((16,), -jnp.inf, jnp.float32)
    for q in range(SLOP // 16):
        cand_v[pl.ds(q * 16, 16)] = neg_inf
        cand_i[pl.ds(q * 16, 16)] = zeros

    def vec2(v, o):
        sv = chunk_s[pl.ds(v * 16, 16)]
        mv = chunk_m[pl.ds(v * 16, 16)]
        ms = sv * mv
        key = _ordered_key(ms)
        sel = key >= tkey
        gidx = lane + (base + v * 16)
        plsc.store_compressed(cand_v.at[pl.ds(o, 16)], ms, mask=sel)
        plsc.store_compressed(cand_i.at[pl.ds(o, 16)], gidx, mask=sel)
        cnt = jnp.max(plsc.all_reduce_population_count(sel))
        return jnp.minimum(o + cnt, CAP)
    count = lax.fori_loop(0, VECS, vec2, jnp.int32(0))

    w = _wid()
    cnt_vec[pl.ds(0, 16)] = zeros + count
    pltpu.sync_copy(cand_v.at[pl.ds(0, ROW)], vals_out.at[w])
    pltpu.sync_copy(cand_i.at[pl.ds(0, ROW)], idx_out.at[w])
    pltpu.sync_copy(cnt_vec, cnt_out.at[w])


_k2_call = pl.kernel(
    _k2_body,
    out_type=(
        jax.ShapeDtypeStruct((WORKERS, ROW), jnp.float32),
        jax.ShapeDtypeStruct((WORKERS, ROW), jnp.int32),
        jax.ShapeDtypeStruct((WORKERS, 16), jnp.int32),
    ),
    mesh=plsc.VectorSubcoreMesh(
        core_axis_name="c", subcore_axis_name="s", num_cores=2),
    compiler_params=pltpu.CompilerParams(needs_layout_passes=False),
    scratch_types=[
        pltpu.VMEM((2 * BINS,), jnp.int32),
        pltpu.VMEM((PER_W,), jnp.float32),
        pltpu.VMEM((PER_W,), jnp.float32),
        pltpu.VMEM((SLOP,), jnp.float32),
        pltpu.VMEM((SLOP,), jnp.int32),
        pltpu.VMEM((16,), jnp.int32),
    ],
)


TCR, TCC = 16, 128


def _xorshuf(x, d):
    if d < TCC:
        fwd = jnp.roll(x, -d, axis=1)
        bwd = jnp.roll(x, d, axis=1)
        col = lax.broadcasted_iota(jnp.int32, (TCR, TCC), 1)
        take_fwd = (col & d) == 0
    else:
        r = d // TCC
        fwd = jnp.roll(x, -r, axis=0)
        bwd = jnp.roll(x, r, axis=0)
        row = lax.broadcasted_iota(jnp.int32, (TCR, TCC), 0)
        take_fwd = (row & r) == 0
    return jnp.where(take_fwd, fwd, bwd)


def _tc_body(vals_ref, idx_ref, cnt2d_ref, ov_ref, oi_ref):
    vals = vals_ref[...]
    idx = idx_ref[...]
    cnt2d = cnt2d_ref[...]
    col = lax.broadcasted_iota(jnp.int32, (TCR, TCC), 1)
    row = lax.broadcasted_iota(jnp.int32, (TCR, TCC), 0)
    valid = (col & (CAP - 1)) < cnt2d
    v = jnp.where(valid, vals, -jnp.inf)
    flat = row * TCC + col
    pos = flat
    k = 2
    while k <= C:
        dirm = (flat & k) == 0
        j = k // 2
        while j >= 1:
            pv = _xorshuf(v, j)
            pp = _xorshuf(pos, j)
            pi = _xorshuf(idx, j)
            am_first = (flat & j) == 0
            p_ahead = (pv > v) | ((pv == v) & (pp < pos))
            keep_self = (dirm != p_ahead) == am_first
            v = jnp.where(keep_self, v, pv)
            pos = jnp.where(keep_self, pos, pp)
            idx = jnp.where(keep_self, idx, pi)
            j //= 2
        k *= 2
    ov_ref[...] = v[0:TOPK // TCC, :]
    oi_ref[...] = idx[0:TOPK // TCC, :]


_tc_call = pl.pallas_call(
    _tc_body,
    out_shape=(
        jax.ShapeDtypeStruct((TOPK // TCC, TCC), jnp.float32),
        jax.ShapeDtypeStruct((TOPK // TCC, TCC), jnp.int32),
    ),
)


def kernel(influence_scores, icv_mask):
    maskf = icv_mask.astype(jnp.float32)
    pad = NPAD - NIN
    s = jnp.concatenate([influence_scores,
                         jnp.zeros((pad,), jnp.float32)])
    mf = jnp.concatenate([maskf, jnp.zeros((pad,), jnp.float32)])
    ghist = _k1_call(s, mf)
    cand_v, cand_i, cand_c = _k2_call(s, mf, ghist)
    v16 = cand_v[:, :CAP].reshape(TCR, TCC)
    i16 = cand_i[:, :CAP].reshape(TCR, TCC)
    cnt2d = jnp.repeat(cand_c[:, 0].reshape(TCR, TCC // CAP), CAP, axis=1)
    ov, oi = _tc_call(v16, i16, cnt2d)
    return oi.reshape(TOPK), ov.reshape(TOPK)


# speedup vs baseline: 1.0673x; 1.0673x over previous
import jax
import jax.numpy as jnp
from jax import lax
from jax.experimental import pallas as pl
from jax.experimental.pallas import tpu as pltpu
from jax.experimental.pallas import tpu_sc as plsc

TOPK = 256
NIN = 1_000_000
WORKERS = 32
PER_W = 31_264
NPAD = WORKERS * PER_W
VECS = PER_W // 16
BINS = 2048
SHIFT = 21
CAP = 64
ROW = 128
SLOP = ROW + 16
C = WORKERS * CAP


def _ordered_key(v):
    b = plsc.bitcast(v, jnp.uint32)
    neg = b >= jnp.uint32(0x80000000)
    return jnp.where(neg, ~b, b | jnp.uint32(0x80000000))


def _wid():
    return lax.axis_index("c") * 16 + lax.axis_index("s")


def _k1_body(scores_hbm, maskf_hbm, ghist_out,
             hist, merged, colsum, chunk_s, chunk_m, sem, sp_hist):
    c = lax.axis_index("c")
    s = lax.axis_index("s")
    base = _wid() * PER_W
    lane = lax.iota(jnp.int32, 16)
    ones = jnp.ones((16,), jnp.int32)
    zeros = jnp.zeros((16,), jnp.int32)

    h1 = pltpu.async_copy(scores_hbm.at[pl.ds(base, PER_W)], chunk_s, sem)
    h2 = pltpu.async_copy(maskf_hbm.at[pl.ds(base, PER_W)], chunk_m, sem)

    def zero_j(j, _):
        for l in range(16):
            hist[l, pl.ds(j * 16, 16)] = zeros
        return 0
    lax.fori_loop(0, BINS // 16, zero_j, 0)
    h1.wait()
    h2.wait()

    def histo(v):
        sv = chunk_s[pl.ds(v * 16, 16)]
        mv = chunk_m[pl.ds(v * 16, 16)]
        key = _ordered_key(sv * mv)
        bins = lax.convert_element_type(key >> jnp.uint32(SHIFT), jnp.int32)
        plsc.addupdate_scatter(hist, [lane, bins], ones)

    def vec1(v, _):
        for u in range(4):
            histo(v * 4 + u)
        return 0
    lax.fori_loop(0, VECS // 4, vec1, 0)
    for v in range(VECS - VECS % 4, VECS):
        histo(v)

    def merge_j(j, _):
        acc = hist[0, pl.ds(j * 16, 16)]
        for l in range(1, 16):
            acc = acc + hist[l, pl.ds(j * 16, 16)]
        merged[pl.ds(j * 16, 16)] = acc
        return 0
    lax.fori_loop(0, BINS // 16, merge_j, 0)
    pltpu.sync_copy(merged, sp_hist.at[s])
    plsc.subcore_barrier()

    CB = BINS // 16
    pltpu.sync_copy(sp_hist.at[:, pl.ds(s * CB, CB)], colsum)

    def col_j(j, _):
        acc = colsum[0, pl.ds(j * 16, 16)]
        for l in range(1, 16):
            acc = acc + colsum[l, pl.ds(j * 16, 16)]
        merged[pl.ds(j * 16, 16)] = acc
        return 0
    lax.fori_loop(0, CB // 16, col_j, 0)
    pltpu.sync_copy(merged.at[pl.ds(0, CB)],
                    ghist_out.at[pl.ds(c * BINS + s * CB, CB)])


_k1_call = pl.kernel(
    _k1_body,
    out_type=jax.ShapeDtypeStruct((2 * BINS,), jnp.int32),
    mesh=plsc.VectorSubcoreMesh(
        core_axis_name="c", subcore_axis_name="s", num_cores=2),
    compiler_params=pltpu.CompilerParams(needs_layout_passes=False),
    scratch_types=[
        pltpu.VMEM((16, BINS), jnp.int32),
        pltpu.VMEM((BINS,), jnp.int32),
        pltpu.VMEM((16, BINS // 16), jnp.int32),
        pltpu.VMEM((PER_W,), jnp.float32),
        pltpu.VMEM((PER_W,), jnp.float32),
        pltpu.SemaphoreType.DMA,
        pltpu.VMEM_SHARED((16, BINS), jnp.int32),
    ],
)


def _k2_body(scores_hbm, maskf_hbm, ghist_hbm, vals_out, idx_out, cnt_out,
             ghist, chunk_s, chunk_m, cand_v, cand_i, cnt_vec, sem):
    base = _wid() * PER_W
    lane = lax.iota(jnp.int32, 16)
    zeros = jnp.zeros((16,), jnp.int32)

    h1 = pltpu.async_copy(scores_hbm.at[pl.ds(base, PER_W)], chunk_s, sem)
    h2 = pltpu.async_copy(maskf_hbm.at[pl.ds(base, PER_W)], chunk_m, sem)
    pltpu.sync_copy(ghist_hbm, ghist)

    def scan_i(i, carry):
        acc, done, tkey = carry
        j = (BINS // 16 - 1) - i
        g = (ghist[pl.ds(j * 16, 16)]
             + ghist[pl.ds(BINS + j * 16, 16)])
        rev = lax.rev(g, (0,))
        cs = lax.cumsum(rev)
        total = jnp.max(cs)
        ge = (acc + cs) >= TOPK
        f = jnp.max(plsc.all_reduce_ffs(ge))
        tj = lax.convert_element_type(j * 16 + 15 - f, jnp.uint32)
        tj = tj << jnp.uint32(SHIFT)
        crossed = (acc + total) >= TOPK
        first = jnp.logical_and(jnp.logical_not(done), crossed)
        tkey = jnp.where(first, tj, tkey)
        done = jnp.logical_or(done, crossed)
        return acc + total, done, tkey
    _, _, tkey = lax.fori_loop(
        0, BINS // 16, scan_i,
        (jnp.int32(0), jnp.bool_(False), jnp.uint32(0)))
    h1.wait()
    h2.wait()

    neg_inf = jnp.full((16,), -jnp.inf, jnp.float32)
    for q in range(SLOP // 16):
        cand_v[pl.ds(q * 16, 16)] = neg_inf
        cand_i[pl.ds(q * 16, 16)] = zeros

    def compact(v, o):
        sv = chunk_s[pl.ds(v * 16, 16)]
        mv = chunk_m[pl.ds(v * 16, 16)]
        ms = sv * mv
        key = _ordered_key(ms)
        sel = key >= tkey
        gidx = lane + (base + v * 16)
        plsc.store_compressed(cand_v.at[pl.ds(o, 16)], ms, mask=sel)
        plsc.store_compressed(cand_i.at[pl.ds(o, 16)], gidx, mask=sel)
        cnt = jnp.max(plsc.all_reduce_population_count(sel))
        return jnp.minimum(o + cnt, CAP)

    def vec2(v, o):
        o = compact(v * 2, o)
        return compact(v * 2 + 1, o)
    count = lax.fori_loop(0, VECS // 2, vec2, jnp.int32(0))

    w = _wid()
    cnt_vec[pl.ds(0, 16)] = zeros + count
    pltpu.sync_copy(cand_v.at[pl.ds(0, ROW)], vals_out.at[w])
    pltpu.sync_copy(cand_i.at[pl.ds(0, ROW)], idx_out.at[w])
    pltpu.sync_copy(cnt_vec, cnt_out.at[w])


_k2_call = pl.kernel(
    _k2_body,
    out_type=(
        jax.ShapeDtypeStruct((WORKERS, ROW), jnp.float32),
        jax.ShapeDtypeStruct((WORKERS, ROW), jnp.int32),
        jax.ShapeDtypeStruct((WORKERS, 16), jnp.int32),
    ),
    mesh=plsc.VectorSubcoreMesh(
        core_axis_name="c", subcore_axis_name="s", num_cores=2),
    compiler_params=pltpu.CompilerParams(needs_layout_passes=False),
    scratch_types=[
        pltpu.VMEM((2 * BINS,), jnp.int32),
        pltpu.VMEM((PER_W,), jnp.float32),
        pltpu.VMEM((PER_W,), jnp.float32),
        pltpu.VMEM((SLOP,), jnp.float32),
        pltpu.VMEM((SLOP,), jnp.int32),
        pltpu.VMEM((16,), jnp.int32),
        pltpu.SemaphoreType.DMA,
    ],
)


TCR, TCC = 16, 128


def _xorshuf(x, d):
    if d < TCC:
        fwd = jnp.roll(x, -d, axis=1)
        bwd = jnp.roll(x, d, axis=1)
        col = lax.broadcasted_iota(jnp.int32, (TCR, TCC), 1)
        take_fwd = (col & d) == 0
    else:
        r = d // TCC
        fwd = jnp.roll(x, -r, axis=0)
        bwd = jnp.roll(x, r, axis=0)
        row = lax.broadcasted_iota(jnp.int32, (TCR, TCC), 0)
        take_fwd = (row & r) == 0
    return jnp.where(take_fwd, fwd, bwd)


def _tc_body(vals_ref, idx_ref, cnt2d_ref, ov_ref, oi_ref):
    vals = vals_ref[...]
    idx = idx_ref[...]
    cnt2d = cnt2d_ref[...]
    col = lax.broadcasted_iota(jnp.int32, (TCR, TCC), 1)
    row = lax.broadcasted_iota(jnp.int32, (TCR, TCC), 0)
    valid = (col & (CAP - 1)) < cnt2d
    v = jnp.where(valid, vals, -jnp.inf)
    flat = row * TCC + col
    pos = flat
    k = 2
    while k <= C:
        dirm = (flat & k) == 0
        j = k // 2
        while j >= 1:
            pv = _xorshuf(v, j)
            pp = _xorshuf(pos, j)
            pi = _xorshuf(idx, j)
            am_first = (flat & j) == 0
            p_ahead = (pv > v) | ((pv == v) & (pp < pos))
            keep_self = (dirm != p_ahead) == am_first
            v = jnp.where(keep_self, v, pv)
            pos = jnp.where(keep_self, pos, pp)
            idx = jnp.where(keep_self, idx, pi)
            j //= 2
        k *= 2
    ov_ref[...] = v[0:TOPK // TCC, :]
    oi_ref[...] = idx[0:TOPK // TCC, :]


_tc_call = pl.pallas_call(
    _tc_body,
    out_shape=(
        jax.ShapeDtypeStruct((TOPK // TCC, TCC), jnp.float32),
        jax.ShapeDtypeStruct((TOPK // TCC, TCC), jnp.int32),
    ),
)


def kernel(influence_scores, icv_mask):
    maskf = icv_mask.astype(jnp.float32)
    pad = NPAD - NIN
    s = jnp.concatenate([influence_scores,
                         jnp.zeros((pad,), jnp.float32)])
    mf = jnp.concatenate([maskf, jnp.zeros((pad,), jnp.float32)])
    ghist = _k1_call(s, mf)
    cand_v, cand_i, cand_c = _k2_call(s, mf, ghist)
    v16 = cand_v[:, :CAP].reshape(TCR, TCC)
    i16 = cand_i[:, :CAP].reshape(TCR, TCC)
    cnt2d = jnp.repeat(cand_c[:, 0].reshape(TCR, TCC // CAP), CAP, axis=1)
    ov, oi = _tc_call(v16, i16, cnt2d)
    return oi.reshape(TOPK), ov.reshape(TOPK)


# speedup vs baseline: 1.0952x; 1.0261x over previous
import jax
import jax.numpy as jnp
from jax import lax
from jax.experimental import pallas as pl
from jax.experimental.pallas import tpu as pltpu
from jax.experimental.pallas import tpu_sc as plsc

TOPK = 256
NIN = 1_000_000
WORKERS = 32
PER_W = 31_248
LAST_W = NIN - 31 * PER_W
BINS = 2048
SHIFT = 21
CAP = 64
ROW = 128
SLOP = ROW + 16
C = WORKERS * CAP


def _ordered_key(v):
    b = plsc.bitcast(v, jnp.uint32)
    neg = b >= jnp.uint32(0x80000000)
    return jnp.where(neg, ~b, b | jnp.uint32(0x80000000))


def _wid():
    return lax.axis_index("c") * 16 + lax.axis_index("s")


def _k1_body(scores_hbm, maskf_hbm, ghist_out,
             hist, merged, colsum, chunk_s, chunk_m, sem, sp_hist):
    c = lax.axis_index("c")
    s = lax.axis_index("s")
    w = _wid()
    base = w * PER_W
    is_last = w == WORKERS - 1
    lane = lax.iota(jnp.int32, 16)
    ones = jnp.ones((16,), jnp.int32)
    zeros = jnp.zeros((16,), jnp.int32)

    h1 = pltpu.async_copy(scores_hbm.at[pl.ds(base, PER_W)],
                          chunk_s.at[pl.ds(0, PER_W)], sem)
    h2 = pltpu.async_copy(maskf_hbm.at[pl.ds(base, PER_W)],
                          chunk_m.at[pl.ds(0, PER_W)], sem)

    @pl.when(is_last)
    def _():
        pltpu.sync_copy(scores_hbm.at[pl.ds(NIN - 64, 64)],
                        chunk_s.at[pl.ds(PER_W, 64)])
        pltpu.sync_copy(maskf_hbm.at[pl.ds(NIN - 64, 64)],
                        chunk_m.at[pl.ds(PER_W, 64)])

    def zero_j(j, _):
        for l in range(16):
            hist[l, pl.ds(j * 16, 16)] = zeros
        return 0
    lax.fori_loop(0, BINS // 16, zero_j, 0)
    h1.wait()
    h2.wait()

    def histo(v):
        sv = chunk_s[pl.ds(v * 16, 16)]
        mv = chunk_m[pl.ds(v * 16, 16)]
        key = _ordered_key(sv * mv)
        bins = lax.convert_element_type(key >> jnp.uint32(SHIFT), jnp.int32)
        plsc.addupdate_scatter(hist, [lane, bins], ones)

    vecs4 = jnp.where(is_last, LAST_W // 16 // 4, PER_W // 16 // 4)

    def vec1(v, _):
        for u in range(4):
            histo(v * 4 + u)
        return 0
    lax.fori_loop(0, vecs4, vec1, 0)
    histo(vecs4 * 4)

    def merge_j(j, _):
        acc = hist[0, pl.ds(j * 16, 16)]
        for l in range(1, 16):
            acc = acc + hist[l, pl.ds(j * 16, 16)]
        merged[pl.ds(j * 16, 16)] = acc
        return 0
    lax.fori_loop(0, BINS // 16, merge_j, 0)
    pltpu.sync_copy(merged, sp_hist.at[s])
    plsc.subcore_barrier()

    CB = BINS // 16
    pltpu.sync_copy(sp_hist.at[:, pl.ds(s * CB, CB)], colsum)

    def col_j(j, _):
        acc = colsum[0, pl.ds(j * 16, 16)]
        for l in range(1, 16):
            acc = acc + colsum[l, pl.ds(j * 16, 16)]
        merged[pl.ds(j * 16, 16)] = acc
        return 0
    lax.fori_loop(0, CB // 16, col_j, 0)
    pltpu.sync_copy(merged.at[pl.ds(0, CB)],
                    ghist_out.at[pl.ds(c * BINS + s * CB, CB)])


_k1_call = pl.kernel(
    _k1_body,
    out_type=jax.ShapeDtypeStruct((2 * BINS,), jnp.int32),
    mesh=plsc.VectorSubcoreMesh(
        core_axis_name="c", subcore_axis_name="s", num_cores=2),
    compiler_params=pltpu.CompilerParams(needs_layout_passes=False),
    scratch_types=[
        pltpu.VMEM((16, BINS), jnp.int32),
        pltpu.VMEM((BINS,), jnp.int32),
        pltpu.VMEM((16, BINS // 16), jnp.int32),
        pltpu.VMEM((LAST_W,), jnp.float32),
        pltpu.VMEM((LAST_W,), jnp.float32),
        pltpu.SemaphoreType.DMA,
        pltpu.VMEM_SHARED((16, BINS), jnp.int32),
    ],
)


def _k2_body(scores_hbm, maskf_hbm, ghist_hbm, vals_out, idx_out, cnt_out,
             ghist, chunk_s, chunk_m, cand_v, cand_i, cnt_vec, sem):
    w = _wid()
    base = w * PER_W
    is_last = w == WORKERS - 1
    lane = lax.iota(jnp.int32, 16)
    zeros = jnp.zeros((16,), jnp.int32)

    h1 = pltpu.async_copy(scores_hbm.at[pl.ds(base, PER_W)],
                          chunk_s.at[pl.ds(0, PER_W)], sem)
    h2 = pltpu.async_copy(maskf_hbm.at[pl.ds(base, PER_W)],
                          chunk_m.at[pl.ds(0, PER_W)], sem)

    @pl.when(is_last)
    def _():
        pltpu.sync_copy(scores_hbm.at[pl.ds(NIN - 64, 64)],
                        chunk_s.at[pl.ds(PER_W, 64)])
        pltpu.sync_copy(maskf_hbm.at[pl.ds(NIN - 64, 64)],
                        chunk_m.at[pl.ds(PER_W, 64)])
    pltpu.sync_copy(ghist_hbm, ghist)

    def scan_i(i, carry):
        acc, done, tkey = carry
        j = (BINS // 16 - 1) - i
        g = (ghist[pl.ds(j * 16, 16)]
             + ghist[pl.ds(BINS + j * 16, 16)])
        rev = lax.rev(g, (0,))
        cs = lax.cumsum(rev)
        total = jnp.max(cs)
        ge = (acc + cs) >= TOPK
        f = jnp.max(plsc.all_reduce_ffs(ge))
        tj = lax.convert_element_type(j * 16 + 15 - f, jnp.uint32)
        tj = tj << jnp.uint32(SHIFT)
        crossed = (acc + total) >= TOPK
        first = jnp.logical_and(jnp.logical_not(done), crossed)
        tkey = jnp.where(first, tj, tkey)
        done = jnp.logical_or(done, crossed)
        return acc + total, done, tkey
    _, _, tkey = lax.fori_loop(
        0, BINS // 16, scan_i,
        (jnp.int32(0), jnp.bool_(False), jnp.uint32(0)))
    h1.wait()
    h2.wait()

    neg_inf = jnp.full((16,), -jnp.inf, jnp.float32)
    for q in range(SLOP // 16):
        cand_v[pl.ds(q * 16, 16)] = neg_inf
        cand_i[pl.ds(q * 16, 16)] = zeros

    def compact(v, o):
        sv = chunk_s[pl.ds(v * 16, 16)]
        mv = chunk_m[pl.ds(v * 16, 16)]
        ms = sv * mv
        key = _ordered_key(ms)
        sel = key >= tkey
        gidx = lane + (base + v * 16)
        plsc.store_compressed(cand_v.at[pl.ds(o, 16)], ms, mask=sel)
        plsc.store_compressed(cand_i.at[pl.ds(o, 16)], gidx, mask=sel)
        cnt = jnp.max(plsc.all_reduce_population_count(sel))
        return jnp.minimum(o + cnt, CAP)

    vecs2 = jnp.where(is_last, LAST_W // 16 // 2, PER_W // 16 // 2)

    def vec2(v, o):
        o = compact(v * 2, o)
        return compact(v * 2 + 1, o)
    count = lax.fori_loop(0, vecs2, vec2, jnp.int32(0))
    count = compact(vecs2 * 2, count)

    cnt_vec[pl.ds(0, 16)] = zeros + count
    pltpu.sync_copy(cand_v.at[pl.ds(0, ROW)], vals_out.at[w])
    pltpu.sync_copy(cand_i.at[pl.ds(0, ROW)], idx_out.at[w])
    pltpu.sync_copy(cnt_vec, cnt_out.at[w])


_k2_call = pl.kernel(
    _k2_body,
    out_type=(
        jax.ShapeDtypeStruct((WORKERS, ROW), jnp.float32),
        jax.ShapeDtypeStruct((WORKERS, ROW), jnp.int32),
        jax.ShapeDtypeStruct((WORKERS, 16), jnp.int32),
    ),
    mesh=plsc.VectorSubcoreMesh(
        core_axis_name="c", subcore_axis_name="s", num_cores=2),
    compiler_params=pltpu.CompilerParams(needs_layout_passes=False),
    scratch_types=[
        pltpu.VMEM((2 * BINS,), jnp.int32),
        pltpu.VMEM((LAST_W,), jnp.float32),
        pltpu.VMEM((LAST_W,), jnp.float32),
        pltpu.VMEM((SLOP,), jnp.float32),
        pltpu.VMEM((SLOP,), jnp.int32),
        pltpu.VMEM((16,), jnp.int32),
        pltpu.SemaphoreType.DMA,
    ],
)


TCR, TCC = 16, 128


def _xorshuf(x, d):
    if d < TCC:
        fwd = jnp.roll(x, -d, axis=1)
        bwd = jnp.roll(x, d, axis=1)
        col = lax.broadcasted_iota(jnp.int32, (TCR, TCC), 1)
        take_fwd = (col & d) == 0
    else:
        r = d // TCC
        fwd = jnp.roll(x, -r, axis=0)
        bwd = jnp.roll(x, r, axis=0)
        row = lax.broadcasted_iota(jnp.int32, (TCR, TCC), 0)
        take_fwd = (row & r) == 0
    return jnp.where(take_fwd, fwd, bwd)


def _tc_body(vals_ref, idx_ref, cnt2d_ref, ov_ref, oi_ref):
    vals = vals_ref[...]
    idx = idx_ref[...]
    cnt2d = cnt2d_ref[...]
    col = lax.broadcasted_iota(jnp.int32, (TCR, TCC), 1)
    row = lax.broadcasted_iota(jnp.int32, (TCR, TCC), 0)
    valid = (col & (CAP - 1)) < cnt2d
    v = jnp.where(valid, vals, -jnp.inf)
    flat = row * TCC + col
    pos = flat
    k = 2
    while k <= C:
        dirm = (flat & k) == 0
        j = k // 2
        while j >= 1:
            pv = _xorshuf(v, j)
            pp = _xorshuf(pos, j)
            pi = _xorshuf(idx, j)
            am_first = (flat & j) == 0
            p_ahead = (pv > v) | ((pv == v) & (pp < pos))
            keep_self = (dirm != p_ahead) == am_first
            v = jnp.where(keep_self, v, pv)
            pos = jnp.where(keep_self, pos, pp)
            idx = jnp.where(keep_self, idx, pi)
            j //= 2
        k *= 2
    ov_ref[...] = v[0:TOPK // TCC, :]
    oi_ref[...] = idx[0:TOPK // TCC, :]


_tc_call = pl.pallas_call(
    _tc_body,
    out_shape=(
        jax.ShapeDtypeStruct((TOPK // TCC, TCC), jnp.float32),
        jax.ShapeDtypeStruct((TOPK // TCC, TCC), jnp.int32),
    ),
)


def kernel(influence_scores, icv_mask):
    s = influence_scores
    mf = icv_mask.astype(jnp.float32)
    ghist = _k1_call(s, mf)
    cand_v, cand_i, cand_c = _k2_call(s, mf, ghist)
    v16 = cand_v[:, :CAP].reshape(TCR, TCC)
    i16 = cand_i[:, :CAP].reshape(TCR, TCC)
    cnt2d = jnp.repeat(cand_c[:, 0].reshape(TCR, TCC // CAP), CAP, axis=1)
    ov, oi = _tc_call(v16, i16, cnt2d)
    return oi.reshape(TOPK), ov.reshape(TOPK)


# speedup vs baseline: 1.2428x; 1.1347x over previous
import jax
import jax.numpy as jnp
from jax import lax
from jax.experimental import pallas as pl
from jax.experimental.pallas import tpu as pltpu
from jax.experimental.pallas import tpu_sc as plsc

TOPK = 256
NIN = 1_000_000
WORKERS = 32
PER_W = 31_248
LAST_W = NIN - 31 * PER_W
BINS = 2048
SHIFT = 21
CAP = 64
ROW = 128
SLOP = ROW + 16
C = WORKERS * CAP


def _ordered_key(v):
    b = plsc.bitcast(v, jnp.uint32)
    neg = b >= jnp.uint32(0x80000000)
    return jnp.where(neg, ~b, b | jnp.uint32(0x80000000))


def _wid():
    return lax.axis_index("c") * 16 + lax.axis_index("s")


def _k1_body(scores_hbm, maskf_hbm, ghist_out,
             hist, merged, colsum, chunk_s, chunk_m, sem, sp_hist):
    c = lax.axis_index("c")
    s = lax.axis_index("s")
    w = _wid()
    base = w * PER_W
    is_last = w == WORKERS - 1
    lane = lax.iota(jnp.int32, 16)
    ones = jnp.ones((16,), jnp.int32)
    zeros = jnp.zeros((16,), jnp.int32)

    h1 = pltpu.async_copy(scores_hbm.at[pl.ds(base, PER_W)],
                          chunk_s.at[pl.ds(0, PER_W)], sem)
    h2 = pltpu.async_copy(maskf_hbm.at[pl.ds(base, PER_W)],
                          chunk_m.at[pl.ds(0, PER_W)], sem)

    @pl.when(is_last)
    def _():
        pltpu.sync_copy(scores_hbm.at[pl.ds(NIN - 64, 64)],
                        chunk_s.at[pl.ds(PER_W, 64)])
        pltpu.sync_copy(maskf_hbm.at[pl.ds(NIN - 64, 64)],
                        chunk_m.at[pl.ds(PER_W, 64)])

    def zero_j(j, _):
        for l in range(16):
            hist[l, pl.ds(j * 16, 16)] = zeros
        return 0
    lax.fori_loop(0, BINS // 16, zero_j, 0)
    h1.wait()
    h2.wait()

    def histo(v):
        sv = chunk_s[pl.ds(v * 16, 16)]
        mv = chunk_m[pl.ds(v * 16, 16)]
        key = _ordered_key(sv * mv)
        bins = lax.convert_element_type(key >> jnp.uint32(SHIFT), jnp.int32)
        plsc.addupdate_scatter(hist, [lane, bins], ones)

    vecs4 = jnp.where(is_last, LAST_W // 16 // 4, PER_W // 16 // 4)

    def vec1(v, _):
        for u in range(4):
            histo(v * 4 + u)
        return 0
    lax.fori_loop(0, vecs4, vec1, 0)
    histo(vecs4 * 4)

    def merge_j(j, _):
        acc = hist[0, pl.ds(j * 16, 16)]
        for l in range(1, 16):
            acc = acc + hist[l, pl.ds(j * 16, 16)]
        merged[pl.ds(j * 16, 16)] = acc
        return 0
    lax.fori_loop(0, BINS // 16, merge_j, 0)
    pltpu.sync_copy(merged, sp_hist.at[s])
    plsc.subcore_barrier()

    CB = BINS // 16
    pltpu.sync_copy(sp_hist.at[:, pl.ds(s * CB, CB)], colsum)

    def col_j(j, _):
        acc = colsum[0, pl.ds(j * 16, 16)]
        for l in range(1, 16):
            acc = acc + colsum[l, pl.ds(j * 16, 16)]
        merged[pl.ds(j * 16, 16)] = acc
        return 0
    lax.fori_loop(0, CB // 16, col_j, 0)
    pltpu.sync_copy(merged.at[pl.ds(0, CB)],
                    ghist_out.at[pl.ds(c * BINS + s * CB, CB)])


_k1_call = pl.kernel(
    _k1_body,
    out_type=jax.ShapeDtypeStruct((2 * BINS,), jnp.int32),
    mesh=plsc.VectorSubcoreMesh(
        core_axis_name="c", subcore_axis_name="s", num_cores=2),
    compiler_params=pltpu.CompilerParams(needs_layout_passes=False),
    scratch_types=[
        pltpu.VMEM((16, BINS), jnp.int32),
        pltpu.VMEM((BINS,), jnp.int32),
        pltpu.VMEM((16, BINS // 16), jnp.int32),
        pltpu.VMEM((LAST_W,), jnp.float32),
        pltpu.VMEM((LAST_W,), jnp.float32),
        pltpu.SemaphoreType.DMA,
        pltpu.VMEM_SHARED((16, BINS), jnp.int32),
    ],
)


def _k2_body(scores_hbm, maskf_hbm, ghist_hbm, vals_out, idx_out, cnt_out,
             ghist, chunk_s, chunk_m, cand_v, cand_i, cnt_vec, sem):
    w = _wid()
    base = w * PER_W
    is_last = w == WORKERS - 1
    lane = lax.iota(jnp.int32, 16)
    zeros = jnp.zeros((16,), jnp.int32)

    h1 = pltpu.async_copy(scores_hbm.at[pl.ds(base, PER_W)],
                          chunk_s.at[pl.ds(0, PER_W)], sem)
    h2 = pltpu.async_copy(maskf_hbm.at[pl.ds(base, PER_W)],
                          chunk_m.at[pl.ds(0, PER_W)], sem)

    @pl.when(is_last)
    def _():
        pltpu.sync_copy(scores_hbm.at[pl.ds(NIN - 64, 64)],
                        chunk_s.at[pl.ds(PER_W, 64)])
        pltpu.sync_copy(maskf_hbm.at[pl.ds(NIN - 64, 64)],
                        chunk_m.at[pl.ds(PER_W, 64)])
    pltpu.sync_copy(ghist_hbm, ghist)

    def scan_i(i, carry):
        acc, done, tkey = carry
        j = (BINS // 16 - 1) - i
        g = (ghist[pl.ds(j * 16, 16)]
             + ghist[pl.ds(BINS + j * 16, 16)])
        rev = lax.rev(g, (0,))
        cs = lax.cumsum(rev)
        total = jnp.max(cs)
        ge = (acc + cs) >= TOPK
        f = jnp.max(plsc.all_reduce_ffs(ge))
        tj = lax.convert_element_type(j * 16 + 15 - f, jnp.uint32)
        tj = tj << jnp.uint32(SHIFT)
        crossed = (acc + total) >= TOPK
        first = jnp.logical_and(jnp.logical_not(done), crossed)
        tkey = jnp.where(first, tj, tkey)
        done = jnp.logical_or(done, crossed)
        return acc + total, done, tkey
    _, _, tkey = lax.fori_loop(
        0, BINS // 16, scan_i,
        (jnp.int32(0), jnp.bool_(False), jnp.uint32(0)))
    h1.wait()
    h2.wait()

    neg_inf = jnp.full((16,), -jnp.inf, jnp.float32)
    for q in range(SLOP // 16):
        cand_v[pl.ds(q * 16, 16)] = neg_inf
        cand_i[pl.ds(q * 16, 16)] = zeros

    def probe_vec(v):
        sv = chunk_s[pl.ds(v * 16, 16)]
        mv = chunk_m[pl.ds(v * 16, 16)]
        ms = sv * mv
        sel = _ordered_key(ms) >= tkey
        gidx = lane + (base + v * 16)
        cnt = jnp.max(plsc.all_reduce_population_count(sel))
        return ms, gidx, sel, cnt

    def emit(p, o):
        ms, gidx, sel, cnt = p
        plsc.store_compressed(cand_v.at[pl.ds(o, 16)], ms, mask=sel)
        plsc.store_compressed(cand_i.at[pl.ds(o, 16)], gidx, mask=sel)
        return jnp.minimum(o + cnt, CAP)

    vecs4 = jnp.where(is_last, LAST_W // 16 // 4, PER_W // 16 // 4)

    def vec2(v, o):
        ps = [probe_vec(v * 4 + u) for u in range(4)]
        for p in ps:
            o = emit(p, o)
        return o
    count = lax.fori_loop(0, vecs4, vec2, jnp.int32(0))
    count = emit(probe_vec(vecs4 * 4), count)

    cnt_vec[pl.ds(0, 16)] = zeros + count
    pltpu.sync_copy(cand_v.at[pl.ds(0, ROW)], vals_out.at[w])
    pltpu.sync_copy(cand_i.at[pl.ds(0, ROW)], idx_out.at[w])
    pltpu.sync_copy(cnt_vec, cnt_out.at[w])


_k2_call = pl.kernel(
    _k2_body,
    out_type=(
        jax.ShapeDtypeStruct((WORKERS, ROW), jnp.float32),
        jax.ShapeDtypeStruct((WORKERS, ROW), jnp.int32),
        jax.ShapeDtypeStruct((WORKERS, 16), jnp.int32),
    ),
    mesh=plsc.VectorSubcoreMesh(
        core_axis_name="c", subcore_axis_name="s", num_cores=2),
    compiler_params=pltpu.CompilerParams(needs_layout_passes=False),
    scratch_types=[
        pltpu.VMEM((2 * BINS,), jnp.int32),
        pltpu.VMEM((LAST_W,), jnp.float32),
        pltpu.VMEM((LAST_W,), jnp.float32),
        pltpu.VMEM((SLOP,), jnp.float32),
        pltpu.VMEM((SLOP,), jnp.int32),
        pltpu.VMEM((16,), jnp.int32),
        pltpu.SemaphoreType.DMA,
    ],
)


TCR, TCC = 16, 128


def _xorshuf(x, d):
    if d < TCC:
        fwd = jnp.roll(x, -d, axis=1)
        bwd = jnp.roll(x, d, axis=1)
        col = lax.broadcasted_iota(jnp.int32, (TCR, TCC), 1)
        take_fwd = (col & d) == 0
    else:
        r = d // TCC
        fwd = jnp.roll(x, -r, axis=0)
        bwd = jnp.roll(x, r, axis=0)
        row = lax.broadcasted_iota(jnp.int32, (TCR, TCC), 0)
        take_fwd = (row & r) == 0
    return jnp.where(take_fwd, fwd, bwd)


def _tc_body(vals_ref, idx_ref, cnt2d_ref, ov_ref, oi_ref):
    vals = vals_ref[...]
    idx = idx_ref[...]
    cnt2d = cnt2d_ref[...]
    col = lax.broadcasted_iota(jnp.int32, (TCR, TCC), 1)
    row = lax.broadcasted_iota(jnp.int32, (TCR, TCC), 0)
    valid = (col & (CAP - 1)) < cnt2d
    v = jnp.where(valid, vals, -jnp.inf)
    flat = row * TCC + col
    pos = flat
    k = 2
    while k <= C:
        dirm = (flat & k) == 0
        j = k // 2
        while j >= 1:
            pv = _xorshuf(v, j)
            pp = _xorshuf(pos, j)
            pi = _xorshuf(idx, j)
            am_first = (flat & j) == 0
            p_ahead = (pv > v) | ((pv == v) & (pp < pos))
            keep_self = (dirm != p_ahead) == am_first
            v = jnp.where(keep_self, v, pv)
            pos = jnp.where(keep_self, pos, pp)
            idx = jnp.where(keep_self, idx, pi)
            j //= 2
        k *= 2
    ov_ref[...] = v[0:TOPK // TCC, :]
    oi_ref[...] = idx[0:TOPK // TCC, :]


_tc_call = pl.pallas_call(
    _tc_body,
    out_shape=(
        jax.ShapeDtypeStruct((TOPK // TCC, TCC), jnp.float32),
        jax.ShapeDtypeStruct((TOPK // TCC, TCC), jnp.int32),
    ),
)


def kernel(influence_scores, icv_mask):
    s = influence_scores
    mf = icv_mask.astype(jnp.float32)
    ghist = _k1_call(s, mf)
    cand_v, cand_i, cand_c = _k2_call(s, mf, ghist)
    v16 = cand_v[:, :CAP].reshape(TCR, TCC)
    i16 = cand_i[:, :CAP].reshape(TCR, TCC)
    cnt2d = jnp.repeat(cand_c[:, 0].reshape(TCR, TCC // CAP), CAP, axis=1)
    ov, oi = _tc_call(v16, i16, cnt2d)
    return oi.reshape(TOPK), ov.reshape(TOPK)


# speedup vs baseline: 1.4763x; 1.1879x over previous
import jax
import jax.numpy as jnp
from jax import lax
from jax.experimental import pallas as pl
from jax.experimental.pallas import tpu as pltpu
from jax.experimental.pallas import tpu_sc as plsc

TOPK = 256
NIN = 1_000_000
WORKERS = 32
PER_W = 31_248
LAST_W = NIN - 31 * PER_W
BINS = 2048
SHIFT = 21
CAP = 64
ROW = 128
SLOP = ROW + 16
C = WORKERS * CAP


def _bin_of(v):
    b = plsc.bitcast(v, jnp.int32)
    q = lax.shift_right_logical(b & jnp.int32(0x7FFFFFFF), jnp.int32(SHIFT))
    return jnp.where(b < 0, 1023 - q, 1024 + q)


def _wid():
    return lax.axis_index("c") * 16 + lax.axis_index("s")


def _k1_body(scores_hbm, maskf_hbm, ghist_out,
             hist, merged, colsum, chunk_s, chunk_m, sem, sp_hist):
    c = lax.axis_index("c")
    s = lax.axis_index("s")
    w = _wid()
    base = w * PER_W
    is_last = w == WORKERS - 1
    lane = lax.iota(jnp.int32, 16)
    ones = jnp.ones((16,), jnp.int32)
    zeros = jnp.zeros((16,), jnp.int32)

    h1 = pltpu.async_copy(scores_hbm.at[pl.ds(base, PER_W)],
                          chunk_s.at[pl.ds(0, PER_W)], sem)
    h2 = pltpu.async_copy(maskf_hbm.at[pl.ds(base, PER_W)],
                          chunk_m.at[pl.ds(0, PER_W)], sem)

    @pl.when(is_last)
    def _():
        pltpu.sync_copy(scores_hbm.at[pl.ds(NIN - 64, 64)],
                        chunk_s.at[pl.ds(PER_W, 64)])
        pltpu.sync_copy(maskf_hbm.at[pl.ds(NIN - 64, 64)],
                        chunk_m.at[pl.ds(PER_W, 64)])

    def zero_j(j, _):
        for l in range(16):
            hist[l, pl.ds(j * 16, 16)] = zeros
        return 0
    lax.fori_loop(0, BINS // 16, zero_j, 0)
    h1.wait()
    h2.wait()

    def histo_bins(v):
        sv = chunk_s[pl.ds(v * 16, 16)]
        mv = chunk_m[pl.ds(v * 16, 16)]
        return _bin_of(sv * mv)

    vecs4 = jnp.where(is_last, LAST_W // 16 // 4, PER_W // 16 // 4)

    def vec1(v, _):
        bs = [histo_bins(v * 4 + u) for u in range(4)]
        for b in bs:
            plsc.addupdate_scatter(hist, [lane, b], ones)
        return 0
    lax.fori_loop(0, vecs4, vec1, 0)
    plsc.addupdate_scatter(hist, [lane, histo_bins(vecs4 * 4)], ones)

    def merge_j(j, _):
        acc = hist[0, pl.ds(j * 16, 16)]
        for l in range(1, 16):
            acc = acc + hist[l, pl.ds(j * 16, 16)]
        merged[pl.ds(j * 16, 16)] = acc
        return 0
    lax.fori_loop(0, BINS // 16, merge_j, 0)
    pltpu.sync_copy(merged, sp_hist.at[s])
    plsc.subcore_barrier()

    CB = BINS // 16
    pltpu.sync_copy(sp_hist.at[:, pl.ds(s * CB, CB)], colsum)

    def col_j(j, _):
        acc = colsum[0, pl.ds(j * 16, 16)]
        for l in range(1, 16):
            acc = acc + colsum[l, pl.ds(j * 16, 16)]
        merged[pl.ds(j * 16, 16)] = acc
        return 0
    lax.fori_loop(0, CB // 16, col_j, 0)
    pltpu.sync_copy(merged.at[pl.ds(0, CB)],
                    ghist_out.at[pl.ds(c * BINS + s * CB, CB)])


_k1_call = pl.kernel(
    _k1_body,
    out_type=jax.ShapeDtypeStruct((2 * BINS,), jnp.int32),
    mesh=plsc.VectorSubcoreMesh(
        core_axis_name="c", subcore_axis_name="s", num_cores=2),
    compiler_params=pltpu.CompilerParams(needs_layout_passes=False),
    scratch_types=[
        pltpu.VMEM((16, BINS), jnp.int32),
        pltpu.VMEM((BINS,), jnp.int32),
        pltpu.VMEM((16, BINS // 16), jnp.int32),
        pltpu.VMEM((LAST_W,), jnp.float32),
        pltpu.VMEM((LAST_W,), jnp.float32),
        pltpu.SemaphoreType.DMA,
        pltpu.VMEM_SHARED((16, BINS), jnp.int32),
    ],
)


def _k2_body(scores_hbm, maskf_hbm, ghist_hbm, vals_out, idx_out, cnt_out,
             ghist, chunk_s, chunk_m, cand_v, cand_i, cnt_vec, sem):
    w = _wid()
    base = w * PER_W
    is_last = w == WORKERS - 1
    lane = lax.iota(jnp.int32, 16)
    zeros = jnp.zeros((16,), jnp.int32)

    h1 = pltpu.async_copy(scores_hbm.at[pl.ds(base, PER_W)],
                          chunk_s.at[pl.ds(0, PER_W)], sem)
    h2 = pltpu.async_copy(maskf_hbm.at[pl.ds(base, PER_W)],
                          chunk_m.at[pl.ds(0, PER_W)], sem)

    @pl.when(is_last)
    def _():
        pltpu.sync_copy(scores_hbm.at[pl.ds(NIN - 64, 64)],
                        chunk_s.at[pl.ds(PER_W, 64)])
        pltpu.sync_copy(maskf_hbm.at[pl.ds(NIN - 64, 64)],
                        chunk_m.at[pl.ds(PER_W, 64)])
    pltpu.sync_copy(ghist_hbm, ghist)

    def scan_i(i, carry):
        acc, done, tkey = carry
        j = (BINS // 16 - 1) - i
        g = (ghist[pl.ds(j * 16, 16)]
             + ghist[pl.ds(BINS + j * 16, 16)])
        rev = lax.rev(g, (0,))
        cs = lax.cumsum(rev)
        total = jnp.max(cs)
        ge = (acc + cs) >= TOPK
        f = jnp.max(plsc.all_reduce_ffs(ge))
        tj = j * 16 + 15 - f
        crossed = (acc + total) >= TOPK
        first = jnp.logical_and(jnp.logical_not(done), crossed)
        tkey = jnp.where(first, tj, tkey)
        done = jnp.logical_or(done, crossed)
        return acc + total, done, tkey
    _, _, tbin = lax.fori_loop(
        0, BINS // 16, scan_i,
        (jnp.int32(0), jnp.bool_(False), jnp.int32(0)))
    h1.wait()
    h2.wait()

    neg_inf = jnp.full((16,), -jnp.inf, jnp.float32)
    for q in range(SLOP // 16):
        cand_v[pl.ds(q * 16, 16)] = neg_inf
        cand_i[pl.ds(q * 16, 16)] = zeros

    def probe_vec(v):
        sv = chunk_s[pl.ds(v * 16, 16)]
        mv = chunk_m[pl.ds(v * 16, 16)]
        ms = sv * mv
        sel = _bin_of(ms) >= tbin
        gidx = lane + (base + v * 16)
        cnt = jnp.max(plsc.all_reduce_population_count(sel))
        return ms, gidx, sel, cnt

    def emit(p, o):
        ms, gidx, sel, cnt = p
        plsc.store_compressed(cand_v.at[pl.ds(o, 16)], ms, mask=sel)
        plsc.store_compressed(cand_i.at[pl.ds(o, 16)], gidx, mask=sel)
        return jnp.minimum(o + cnt, CAP)

    vecs4 = jnp.where(is_last, LAST_W // 16 // 4, PER_W // 16 // 4)

    def vec2(v, o):
        ps = [probe_vec(v * 4 + u) for u in range(4)]
        for p in ps:
            o = emit(p, o)
        return o
    count = lax.fori_loop(0, vecs4, vec2, jnp.int32(0))
    count = emit(probe_vec(vecs4 * 4), count)

    cnt_vec[pl.ds(0, 16)] = zeros + count
    pltpu.sync_copy(cand_v.at[pl.ds(0, ROW)], vals_out.at[w])
    pltpu.sync_copy(cand_i.at[pl.ds(0, ROW)], idx_out.at[w])
    pltpu.sync_copy(cnt_vec, cnt_out.at[w])


_k2_call = pl.kernel(
    _k2_body,
    out_type=(
        jax.ShapeDtypeStruct((WORKERS, ROW), jnp.float32),
        jax.ShapeDtypeStruct((WORKERS, ROW), jnp.int32),
        jax.ShapeDtypeStruct((WORKERS, 16), jnp.int32),
    ),
    mesh=plsc.VectorSubcoreMesh(
        core_axis_name="c", subcore_axis_name="s", num_cores=2),
    compiler_params=pltpu.CompilerParams(needs_layout_passes=False),
    scratch_types=[
        pltpu.VMEM((2 * BINS,), jnp.int32),
        pltpu.VMEM((LAST_W,), jnp.float32),
        pltpu.VMEM((LAST_W,), jnp.float32),
        pltpu.VMEM((SLOP,), jnp.float32),
        pltpu.VMEM((SLOP,), jnp.int32),
        pltpu.VMEM((16,), jnp.int32),
        pltpu.SemaphoreType.DMA,
    ],
)


TCR, TCC = 16, 128


def _xorshuf(x, d):
    if d < TCC:
        fwd = jnp.roll(x, -d, axis=1)
        bwd = jnp.roll(x, d, axis=1)
        col = lax.broadcasted_iota(jnp.int32, (TCR, TCC), 1)
        take_fwd = (col & d) == 0
    else:
        r = d // TCC
        fwd = jnp.roll(x, -r, axis=0)
        bwd = jnp.roll(x, r, axis=0)
        row = lax.broadcasted_iota(jnp.int32, (TCR, TCC), 0)
        take_fwd = (row & r) == 0
    return jnp.where(take_fwd, fwd, bwd)


def _tc_body(vals_ref, idx_ref, cnt2d_ref, ov_ref, oi_ref):
    vals = vals_ref[...]
    idx = idx_ref[...]
    cnt2d = cnt2d_ref[...]
    col = lax.broadcasted_iota(jnp.int32, (TCR, TCC), 1)
    row = lax.broadcasted_iota(jnp.int32, (TCR, TCC), 0)
    valid = (col & (CAP - 1)) < cnt2d
    v = jnp.where(valid, vals, -jnp.inf)
    flat = row * TCC + col
    pos = flat
    k = 2
    while k <= C:
        dirm = (flat & k) == 0
        j = k // 2
        while j >= 1:
            pv = _xorshuf(v, j)
            pp = _xorshuf(pos, j)
            pi = _xorshuf(idx, j)
            am_first = (flat & j) == 0
            p_ahead = (pv > v) | ((pv == v) & (pp < pos))
            keep_self = (dirm != p_ahead) == am_first
            v = jnp.where(keep_self, v, pv)
            pos = jnp.where(keep_self, pos, pp)
            idx = jnp.where(keep_self, idx, pi)
            j //= 2
        k *= 2
    ov_ref[...] = v[0:TOPK // TCC, :]
    oi_ref[...] = idx[0:TOPK // TCC, :]


_tc_call = pl.pallas_call(
    _tc_body,
    out_shape=(
        jax.ShapeDtypeStruct((TOPK // TCC, TCC), jnp.float32),
        jax.ShapeDtypeStruct((TOPK // TCC, TCC), jnp.int32),
    ),
)


def kernel(influence_scores, icv_mask):
    s = influence_scores
    mf = icv_mask.astype(jnp.float32)
    ghist = _k1_call(s, mf)
    cand_v, cand_i, cand_c = _k2_call(s, mf, ghist)
    v16 = cand_v[:, :CAP].reshape(TCR, TCC)
    i16 = cand_i[:, :CAP].reshape(TCR, TCC)
    cnt2d = jnp.repeat(cand_c[:, 0].reshape(TCR, TCC // CAP), CAP, axis=1)
    ov, oi = _tc_call(v16, i16, cnt2d)
    return oi.reshape(TOPK), ov.reshape(TOPK)


# speedup vs baseline: 1.6548x; 1.1209x over previous
import jax
import jax.numpy as jnp
from jax import lax
from jax.experimental import pallas as pl
from jax.experimental.pallas import tpu as pltpu
from jax.experimental.pallas import tpu_sc as plsc

TOPK = 256
NIN = 1_000_000
WORKERS = 32
PER_W = 31_248
LAST_W = NIN - 31 * PER_W
BINS = 2048
SHIFT = 21
CAP = 64
ROW = 128
SLOP = ROW + 16
C = WORKERS * CAP


def _bin_of(v):
    b = plsc.bitcast(v, jnp.int32)
    q = lax.shift_right_logical(b & jnp.int32(0x7FFFFFFF), jnp.int32(SHIFT))
    return jnp.where(b < 0, 1023 - q, 1024 + q)


def _wid():
    return lax.axis_index("c") * 16 + lax.axis_index("s")


def _k1_body(scores_hbm, maskf_hbm, ghist_out,
             hist, merged, colsum, chunk_s, chunk_m, sem, sp_hist):
    c = lax.axis_index("c")
    s = lax.axis_index("s")
    w = _wid()
    base = w * PER_W
    is_last = w == WORKERS - 1
    lane = lax.iota(jnp.int32, 16)
    ones = jnp.ones((16,), jnp.int32)
    zeros = jnp.zeros((16,), jnp.int32)

    h1 = pltpu.async_copy(scores_hbm.at[pl.ds(base, PER_W)],
                          chunk_s.at[pl.ds(0, PER_W)], sem)
    h2 = pltpu.async_copy(maskf_hbm.at[pl.ds(base, PER_W)],
                          chunk_m.at[pl.ds(0, PER_W)], sem)

    @pl.when(is_last)
    def _():
        pltpu.sync_copy(scores_hbm.at[pl.ds(NIN - 64, 64)],
                        chunk_s.at[pl.ds(PER_W, 64)])
        pltpu.sync_copy(maskf_hbm.at[pl.ds(NIN - 64, 64)],
                        chunk_m.at[pl.ds(PER_W, 64)])

    def zero_j(j, _):
        for l in range(16):
            hist[l, pl.ds(j * 16, 16)] = zeros
        return 0
    lax.fori_loop(0, BINS // 16, zero_j, 0)
    h1.wait()
    h2.wait()

    def histo_bins(v):
        sv = chunk_s[pl.ds(v * 16, 16)]
        mv = chunk_m[pl.ds(v * 16, 16)]
        return _bin_of(sv * mv)

    vecs = jnp.where(is_last, LAST_W // 16, PER_W // 16)
    U = 8

    def vec1(v, _):
        bs = [histo_bins(v * U + u) for u in range(U)]
        for b in bs:
            plsc.addupdate_scatter(hist, [lane, b], ones)
        return 0
    nu = vecs // U
    lax.fori_loop(0, nu, vec1, 0)

    def tail1(v, _):
        plsc.addupdate_scatter(hist, [lane, histo_bins(v)], ones)
        return 0
    lax.fori_loop(nu * U, vecs, tail1, 0)

    def merge_j(j, _):
        acc = hist[0, pl.ds(j * 16, 16)]
        for l in range(1, 16):
            acc = acc + hist[l, pl.ds(j * 16, 16)]
        merged[pl.ds(j * 16, 16)] = acc
        return 0
    lax.fori_loop(0, BINS // 16, merge_j, 0)
    pltpu.sync_copy(merged, sp_hist.at[s])
    plsc.subcore_barrier()

    CB = BINS // 16
    pltpu.sync_copy(sp_hist.at[:, pl.ds(s * CB, CB)], colsum)

    def col_j(j, _):
        acc = colsum[0, pl.ds(j * 16, 16)]
        for l in range(1, 16):
            acc = acc + colsum[l, pl.ds(j * 16, 16)]
        merged[pl.ds(j * 16, 16)] = acc
        return 0
    lax.fori_loop(0, CB // 16, col_j, 0)
    pltpu.sync_copy(merged.at[pl.ds(0, CB)],
                    ghist_out.at[pl.ds(c * BINS + s * CB, CB)])


_k1_call = pl.kernel(
    _k1_body,
    out_type=jax.ShapeDtypeStruct((2 * BINS,), jnp.int32),
    mesh=plsc.VectorSubcoreMesh(
        core_axis_name="c", subcore_axis_name="s", num_cores=2),
    compiler_params=pltpu.CompilerParams(needs_layout_passes=False),
    scratch_types=[
        pltpu.VMEM((16, BINS), jnp.int32),
        pltpu.VMEM((BINS,), jnp.int32),
        pltpu.VMEM((16, BINS // 16), jnp.int32),
        pltpu.VMEM((LAST_W,), jnp.float32),
        pltpu.VMEM((LAST_W,), jnp.float32),
        pltpu.SemaphoreType.DMA,
        pltpu.VMEM_SHARED((16, BINS), jnp.int32),
    ],
)


def _k2_body(scores_hbm, maskf_hbm, ghist_hbm, vals_out, idx_out, cnt_out,
             ghist, chunk_s, chunk_m, cand_v, cand_i, cnt_vec, sem):
    w = _wid()
    base = w * PER_W
    is_last = w == WORKERS - 1
    lane = lax.iota(jnp.int32, 16)
    zeros = jnp.zeros((16,), jnp.int32)

    h1 = pltpu.async_copy(scores_hbm.at[pl.ds(base, PER_W)],
                          chunk_s.at[pl.ds(0, PER_W)], sem)
    h2 = pltpu.async_copy(maskf_hbm.at[pl.ds(base, PER_W)],
                          chunk_m.at[pl.ds(0, PER_W)], sem)

    @pl.when(is_last)
    def _():
        pltpu.sync_copy(scores_hbm.at[pl.ds(NIN - 64, 64)],
                        chunk_s.at[pl.ds(PER_W, 64)])
        pltpu.sync_copy(maskf_hbm.at[pl.ds(NIN - 64, 64)],
                        chunk_m.at[pl.ds(PER_W, 64)])
    pltpu.sync_copy(ghist_hbm, ghist)

    def scan_i(i, carry):
        acc, done, tkey = carry
        j = (BINS // 16 - 1) - i
        g = (ghist[pl.ds(j * 16, 16)]
             + ghist[pl.ds(BINS + j * 16, 16)])
        rev = lax.rev(g, (0,))
        cs = lax.cumsum(rev)
        total = jnp.max(cs)
        ge = (acc + cs) >= TOPK
        f = jnp.max(plsc.all_reduce_ffs(ge))
        tj = j * 16 + 15 - f
        crossed = (acc + total) >= TOPK
        first = jnp.logical_and(jnp.logical_not(done), crossed)
        tkey = jnp.where(first, tj, tkey)
        done = jnp.logical_or(done, crossed)
        return acc + total, done, tkey
    _, _, tbin = lax.fori_loop(
        0, BINS // 16, scan_i,
        (jnp.int32(0), jnp.bool_(False), jnp.int32(0)))
    h1.wait()
    h2.wait()

    neg_inf = jnp.full((16,), -jnp.inf, jnp.float32)
    for q in range(SLOP // 16):
        cand_v[pl.ds(q * 16, 16)] = neg_inf
        cand_i[pl.ds(q * 16, 16)] = zeros

    def probe_vec(v):
        sv = chunk_s[pl.ds(v * 16, 16)]
        mv = chunk_m[pl.ds(v * 16, 16)]
        ms = sv * mv
        sel = _bin_of(ms) >= tbin
        gidx = lane + (base + v * 16)
        cnt = jnp.max(plsc.all_reduce_population_count(sel))
        return ms, gidx, sel, cnt

    def emit(p, o):
        ms, gidx, sel, cnt = p
        plsc.store_compressed(cand_v.at[pl.ds(o, 16)], ms, mask=sel)
        plsc.store_compressed(cand_i.at[pl.ds(o, 16)], gidx, mask=sel)
        return jnp.minimum(o + cnt, CAP)

    vecs = jnp.where(is_last, LAST_W // 16, PER_W // 16)
    U = 8

    def vec2(v, o):
        ps = [probe_vec(v * U + u) for u in range(U)]
        for p in ps:
            o = emit(p, o)
        return o
    nu = vecs // U
    count = lax.fori_loop(0, nu, vec2, jnp.int32(0))
    count = lax.fori_loop(nu * U, vecs,
                          lambda v, o: emit(probe_vec(v), o), count)

    cnt_vec[pl.ds(0, 16)] = zeros + count
    pltpu.sync_copy(cand_v.at[pl.ds(0, ROW)], vals_out.at[w])
    pltpu.sync_copy(cand_i.at[pl.ds(0, ROW)], idx_out.at[w])
    pltpu.sync_copy(cnt_vec, cnt_out.at[w])


_k2_call = pl.kernel(
    _k2_body,
    out_type=(
        jax.ShapeDtypeStruct((WORKERS, ROW), jnp.float32),
        jax.ShapeDtypeStruct((WORKERS, ROW), jnp.int32),
        jax.ShapeDtypeStruct((WORKERS, 16), jnp.int32),
    ),
    mesh=plsc.VectorSubcoreMesh(
        core_axis_name="c", subcore_axis_name="s", num_cores=2),
    compiler_params=pltpu.CompilerParams(needs_layout_passes=False),
    scratch_types=[
        pltpu.VMEM((2 * BINS,), jnp.int32),
        pltpu.VMEM((LAST_W,), jnp.float32),
        pltpu.VMEM((LAST_W,), jnp.float32),
        pltpu.VMEM((SLOP,), jnp.float32),
        pltpu.VMEM((SLOP,), jnp.int32),
        pltpu.VMEM((16,), jnp.int32),
        pltpu.SemaphoreType.DMA,
    ],
)


TCR, TCC = 16, 128


def _xorshuf(x, d):
    if d < TCC:
        fwd = jnp.roll(x, -d, axis=1)
        bwd = jnp.roll(x, d, axis=1)
        col = lax.broadcasted_iota(jnp.int32, (TCR, TCC), 1)
        take_fwd = (col & d) == 0
    else:
        r = d // TCC
        fwd = jnp.roll(x, -r, axis=0)
        bwd = jnp.roll(x, r, axis=0)
        row = lax.broadcasted_iota(jnp.int32, (TCR, TCC), 0)
        take_fwd = (row & r) == 0
    return jnp.where(take_fwd, fwd, bwd)


def _tc_body(vals_ref, idx_ref, cnt2d_ref, ov_ref, oi_ref):
    vals = vals_ref[...]
    idx = idx_ref[...]
    cnt2d = cnt2d_ref[...]
    col = lax.broadcasted_iota(jnp.int32, (TCR, TCC), 1)
    row = lax.broadcasted_iota(jnp.int32, (TCR, TCC), 0)
    valid = (col & (CAP - 1)) < cnt2d
    v = jnp.where(valid, vals, -jnp.inf)
    flat = row * TCC + col
    pos = flat
    k = 2
    while k <= C:
        dirm = (flat & k) == 0
        j = k // 2
        while j >= 1:
            pv = _xorshuf(v, j)
            pp = _xorshuf(pos, j)
            pi = _xorshuf(idx, j)
            am_first = (flat & j) == 0
            p_ahead = (pv > v) | ((pv == v) & (pp < pos))
            keep_self = (dirm != p_ahead) == am_first
            v = jnp.where(keep_self, v, pv)
            pos = jnp.where(keep_self, pos, pp)
            idx = jnp.where(keep_self, idx, pi)
            j //= 2
        k *= 2
    ov_ref[...] = v[0:TOPK // TCC, :]
    oi_ref[...] = idx[0:TOPK // TCC, :]


_tc_call = pl.pallas_call(
    _tc_body,
    out_shape=(
        jax.ShapeDtypeStruct((TOPK // TCC, TCC), jnp.float32),
        jax.ShapeDtypeStruct((TOPK // TCC, TCC), jnp.int32),
    ),
)


def kernel(influence_scores, icv_mask):
    s = influence_scores
    mf = icv_mask.astype(jnp.float32)
    ghist = _k1_call(s, mf)
    cand_v, cand_i, cand_c = _k2_call(s, mf, ghist)
    v16 = cand_v[:, :CAP].reshape(TCR, TCC)
    i16 = cand_i[:, :CAP].reshape(TCR, TCC)
    cnt2d = jnp.repeat(cand_c[:, 0].reshape(TCR, TCC // CAP), CAP, axis=1)
    ov, oi = _tc_call(v16, i16, cnt2d)
    return oi.reshape(TOPK), ov.reshape(TOPK)
